# baseline (speedup 1.0000x reference)
import functools

import jax
import jax.numpy as jnp
from jax import lax
from jax.experimental import pallas as pl
from jax.experimental.pallas import tpu as pltpu

_TOK_BLOCK = 512
_SUB = 4


def _ffn_block(x_ref, tm_ref, w1_ref, b1_ref, w2_ref, b2_ref, out_ref,
               w1t_ref, w2t_ref):
    T, D = x_ref.shape
    Ts = T // _SUB

    @pl.when(pl.program_id(0) == 0)
    def _():
        w1t_ref[...] = w1_ref[...].astype(jnp.bfloat16).T
        w2t_ref[...] = w2_ref[...].astype(jnp.bfloat16).T

    w1t = w1t_ref[...]
    w2t = w2t_ref[...]
    b1 = b1_ref[...]
    b2 = b2_ref[...]
    for s in range(_SUB):
        rows = pl.ds(s * Ts, Ts)
        tm = tm_ref[rows, :]
        thresh = jnp.where(tm == 0, 96,
                 jnp.where(tm == 1, 192,
                 jnp.where(tm == 2, 384, 768)))
        col = lax.broadcasted_iota(jnp.int32, (Ts, D), 1)
        mask = col < thresh
        xm = jnp.where(mask, x_ref[rows, :], 0.0).astype(jnp.bfloat16)
        h = jnp.dot(xm, w1t, preferred_element_type=jnp.float32)
        h = h + b1
        h = 0.5 * h * (1.0 + lax.erf(h * 0.7071067811865476))
        y = jnp.dot(h.astype(jnp.bfloat16), w2t,
                    preferred_element_type=jnp.float32)
        y = y + b2
        out_ref[rows, :] = jnp.where(mask, y, 0.0)


@functools.partial(jax.jit, static_argnames=())
def kernel(x, token_mask, w1, b1, w2, b2):
    B, S, D = x.shape
    H = w1.shape[0]
    N = B * S
    T = _TOK_BLOCK

    xf = x.reshape(N, D)
    tm = token_mask.reshape(N, 1).astype(jnp.int32)
    b1r = b1.reshape(1, H)
    b2r = b2.reshape(1, D)

    grid = (N // T,)
    out = pl.pallas_call(
        _ffn_block,
        grid=grid,
        in_specs=[
            pl.BlockSpec((T, D), lambda i: (i, 0)),
            pl.BlockSpec((T, 1), lambda i: (i, 0)),
            pl.BlockSpec((H, D), lambda i: (0, 0)),
            pl.BlockSpec((1, H), lambda i: (0, 0)),
            pl.BlockSpec((D, H), lambda i: (0, 0)),
            pl.BlockSpec((1, D), lambda i: (0, 0)),
        ],
        out_specs=pl.BlockSpec((T, D), lambda i: (i, 0)),
        out_shape=jax.ShapeDtypeStruct((N, D), x.dtype),
        scratch_shapes=[
            pltpu.VMEM((D, H), jnp.bfloat16),
            pltpu.VMEM((H, D), jnp.bfloat16),
        ],
        compiler_params=pltpu.CompilerParams(
            dimension_semantics=("arbitrary",),
        ),
    )(xf, tm, w1, b1r, w2, b2r)
    return out.reshape(B, S, D)
